# baseline (speedup 1.0000x reference)
import functools

import jax
import jax.numpy as jnp
from jax import lax
from jax.experimental import pallas as pl
from jax.experimental.pallas import tpu as pltpu
from jax.experimental.pallas import tpu_sc as plsc

N_ATOMS = 1_000_000
B_TOTAL = 16384

NC = 2
NS = 16
LANES = 16

H0 = 500_096
H1 = N_ATOMS - H0
DUMP = H0
SP_WORDS = H0 + 128

CHUNK = B_TOTAL // NS
ROWS = 8
COLS = 128
assert ROWS * COLS == CHUNK

CH = 31_232
CH0_LAST = H0 - 15 * CH
CH1_LAST = H1 - 15 * CH
BUF = max(CH, CH0_LAST, CH1_LAST)

_mesh = plsc.VectorSubcoreMesh(
    core_axis_name="c", subcore_axis_name="s", num_cores=NC, num_subcores=NS
)


@functools.partial(
    pl.kernel,
    out_type=jax.ShapeDtypeStruct((1, N_ATOMS), jnp.float32),
    mesh=_mesh,
    scratch_types=[
        pltpu.VMEM_SHARED((SP_WORDS,), jnp.float32),
        pltpu.VMEM((ROWS, COLS), jnp.int32),
        pltpu.VMEM((ROWS, COLS), jnp.int32),
        pltpu.VMEM((ROWS, COLS), jnp.float32),
        pltpu.VMEM((BUF,), jnp.float32),
        pltpu.VMEM((COLS,), jnp.float32),
    ],
)
def _facts_scatter(v0_hbm, idx_hbm, val_hbm, out_hbm,
                   vsh, idx_raw, idx_loc, vals, bufa, zbuf):
    c = lax.axis_index("c")
    s = lax.axis_index("s")
    base = c * H0
    hsize = H0 - c * (H0 - H1)
    off = pl.multiple_of(s * CH, 128)

    def _fire_init(n, hoff):
        pltpu.sync_copy(v0_hbm.at[0, pl.ds(hoff, n)], bufa.at[pl.ds(0, n)])
        pltpu.sync_copy(bufa.at[pl.ds(0, n)], vsh.at[pl.ds(off, n)])

    @pl.when(s < NS - 1)
    def _init_main():
        _fire_init(CH, base + off)

    @pl.when((s == NS - 1) & (c == 0))
    def _init_last0():
        _fire_init(CH0_LAST, 15 * CH)

    @pl.when((s == NS - 1) & (c == 1))
    def _init_last1():
        _fire_init(CH1_LAST, H0 + 15 * CH)

    pltpu.sync_copy(idx_hbm.at[s], idx_raw)
    pltpu.sync_copy(val_hbm.at[s], vals)

    for r in range(ROWS):
        for k in range(COLS // LANES):
            g = idx_raw[r, pl.ds(k * LANES, LANES)]
            local = g - base
            in_range = (local >= 0) & (local < hsize)
            idx_loc[r, pl.ds(k * LANES, LANES)] = jnp.where(in_range, local, DUMP)

    for k in range(COLS // LANES):
        zbuf[pl.ds(k * LANES, LANES)] = jnp.zeros((LANES,), jnp.float32)

    plsc.subcore_barrier()

    for r in range(ROWS):
        pltpu.sync_copy(vals.at[r], vsh.at[idx_loc.at[r]], add=True)

    pltpu.sync_copy(zbuf, vsh.at[idx_loc.at[ROWS - 1]], add=True)

    plsc.subcore_barrier()

    def _writeback(n, hoff):
        pltpu.sync_copy(vsh.at[pl.ds(off, n)], bufa.at[pl.ds(0, n)])
        @pl.when((c == 0) & (s == 0))
        def _bias():
            lane = lax.iota(jnp.int32, LANES)
            head = bufa[pl.ds(0, LANES)]
            bufa[pl.ds(0, LANES)] = head + jnp.where(
                lane == 0, 1.0, 0.0).astype(jnp.float32)
        pltpu.sync_copy(bufa.at[pl.ds(0, n)], out_hbm.at[0, pl.ds(hoff, n)])

    @pl.when(s < NS - 1)
    def _wb_main():
        _writeback(CH, base + off)

    @pl.when((s == NS - 1) & (c == 0))
    def _wb_last0():
        _writeback(CH0_LAST, 15 * CH)

    @pl.when((s == NS - 1) & (c == 1))
    def _wb_last1():
        _writeback(CH1_LAST, H0 + 15 * CH)


@jax.jit
def kernel(V0, val, bk_idx):
    idx = bk_idx.astype(jnp.int32).reshape(NS, ROWS, COLS)
    vals = val.astype(jnp.float32).reshape(NS, ROWS, COLS)
    return _facts_scatter(V0, idx, vals)
